# baseline (speedup 1.0000x reference)
import functools

import jax
import jax.numpy as jnp
from jax import lax
from jax.experimental import pallas as pl
from jax.experimental.pallas import tpu as pltpu
from jax.experimental.pallas import tpu_sc as plsc

LAMBDA_C = 1.0
_L = 16
_NACC = 8
N_SC = 1024
_BLK = 256
_KPAD = 1024


def _sc_partials(feat, label, centers, n_sc):
    B, D = feat.shape
    NC, NS = 2, 16
    NW = NC * NS
    RPW = n_sc // NW
    RSUB = 16
    NSUB = RPW // RSUB
    NBUF = 2

    mesh = plsc.VectorSubcoreMesh(core_axis_name="c", subcore_axis_name="s")

    @functools.partial(
        pl.kernel,
        mesh=mesh,
        out_type=jax.ShapeDtypeStruct((NW, _L), jnp.float32),
        scratch_types=[
            pltpu.VMEM((RPW,), jnp.int32),
            pltpu.VMEM((NBUF, RSUB, D), jnp.float32),
            pltpu.VMEM((NBUF, RSUB, D), jnp.float32),
            pltpu.VMEM((_L,), jnp.float32),
            pltpu.SemaphoreType.DMA,
            pltpu.SemaphoreType.DMA,
            pltpu.SemaphoreType.DMA,
            pltpu.SemaphoreType.DMA,
        ],
    )
    def k(feat_hbm, label_hbm, centers_hbm, out_hbm,
          idx_v, feat_v, crows_v, part_v, sf0, sf1, sc0, sc1):
        wid = lax.axis_index("s") * NC + lax.axis_index("c")
        base = wid * RPW
        sems_f = (sf0, sf1)
        sems_c = (sc0, sc1)
        pltpu.sync_copy(label_hbm.at[pl.ds(base, RPW)], idx_v)

        def issue(s, b):
            row0 = base + s * RSUB
            pltpu.async_copy(feat_hbm.at[pl.ds(row0, RSUB)],
                             feat_v.at[b], sems_f[b])
            pltpu.async_copy(centers_hbm.at[idx_v.at[pl.ds(s * RSUB, RSUB)]],
                             crows_v.at[b], sems_c[b])

        def wait(s, b):
            row0 = base + s * RSUB
            pltpu.make_async_copy(feat_hbm.at[pl.ds(row0, RSUB)],
                                  feat_v.at[b], sems_f[b]).wait()
            pltpu.make_async_copy(
                centers_hbm.at[idx_v.at[pl.ds(s * RSUB, RSUB)]],
                crows_v.at[b], sems_c[b]).wait()

        for b in range(NBUF):
            issue(b, b)

        def compute_sub(b, accs):
            def row_body(r, accs):
                accs = list(accs)
                for c in range(D // _L):
                    f = feat_v[b, r, pl.ds(c * _L, _L)]
                    g = crows_v[b, r, pl.ds(c * _L, _L)]
                    d = f - g
                    j = c % _NACC
                    accs[j] = accs[j] + d * d
                return tuple(accs)
            return lax.fori_loop(0, RSUB, row_body, accs)

        accs = tuple(jnp.zeros((_L,), jnp.float32) for _ in range(_NACC))

        def group_body(g, accs):
            for b in range(NBUF):
                s = g * NBUF + b
                wait(s, b)
                accs = compute_sub(b, accs)

                @pl.when(s + NBUF < NSUB)
                def _():
                    issue(s + NBUF, b)
            return accs

        accs = lax.fori_loop(0, NSUB // NBUF, group_body, accs)

        total = accs[0]
        for j in range(1, _NACC):
            total = total + accs[j]
        part_v[...] = total
        pltpu.sync_copy(part_v, out_hbm.at[wid])

    return k(feat, label, centers)


def _tc_partial(feat, label2d, centers_bf, n_sc):
    B, D = feat.shape
    n_tc = B - n_sc
    G = n_tc // _BLK
    off = n_sc // _BLK

    def body(feat_ref, lab_ref, cen_ref, out_ref):
        i = pl.program_id(0)
        lab = lab_ref[...]
        onehot = (lax.broadcasted_iota(jnp.int32, (_BLK, _KPAD), 1)
                  == lab).astype(jnp.bfloat16)
        exp = lax.dot_general(onehot, cen_ref[...], (((1,), (0,)), ((), ())),
                              preferred_element_type=jnp.float32)
        d = feat_ref[...] - exp
        s = jnp.sum(d * d, axis=(0, 1), keepdims=True)

        @pl.when(i == 0)
        def _():
            out_ref[...] = s

        @pl.when(i > 0)
        def _():
            out_ref[...] += s

    return pl.pallas_call(
        body,
        grid=(G,),
        in_specs=[
            pl.BlockSpec((_BLK, D), lambda i: (i + off, 0)),
            pl.BlockSpec((_BLK, 1), lambda i: (i + off, 0)),
            pl.BlockSpec((_KPAD, D), lambda i: (0, 0)),
        ],
        out_specs=pl.BlockSpec((1, 1), lambda i: (0, 0)),
        out_shape=jax.ShapeDtypeStruct((1, 1), jnp.float32),
    )(feat, label2d, centers_bf)


def kernel(feat, label, centers):
    B, D = feat.shape
    K = centers.shape[0]
    label = label.astype(jnp.int32)
    centers_bf = jnp.pad(
        centers, ((0, _KPAD - K), (0, 0))).astype(jnp.bfloat16)
    parts_sc = _sc_partials(feat, label, centers, N_SC)
    part_tc = _tc_partial(feat, label.reshape(B, 1), centers_bf, N_SC)
    total = jnp.sum(parts_sc) + part_tc[0, 0]
    return LAMBDA_C / 2.0 / B * jnp.sqrt(total)
